# Initial kernel scaffold; baseline (speedup 1.0000x reference)
import jax
import jax.numpy as jnp
from jax.experimental import pallas as pl


def kernel(inputs_embeds, position_table, ln_gamma, ln_beta):
    raise NotImplementedError("write your pallas kernel here")


# speedup vs baseline: 1.9887x; 1.9887x over previous
import jax
import jax.numpy as jnp
from jax.experimental import pallas as pl

_EPS = 1e-12
_BS = 512


def _ln_body(in_ref, pos_ref, g_ref, b_ref, out_ref):
    x = in_ref[0] + pos_ref[...]
    mean = jnp.mean(x, axis=-1, keepdims=True)
    xc = x - mean
    var = jnp.mean(xc * xc, axis=-1, keepdims=True)
    normed = xc * jax.lax.rsqrt(var + _EPS)
    out_ref[0] = normed * g_ref[0] + b_ref[0]


def kernel(inputs_embeds, position_table, ln_gamma, ln_beta):
    B, S, H = inputs_embeds.shape
    grid = (S // _BS, B)
    return pl.pallas_call(
        _ln_body,
        grid=grid,
        in_specs=[
            pl.BlockSpec((1, _BS, H), lambda i, b: (b, i, 0)),
            pl.BlockSpec((_BS, H), lambda i, b: (i, 0)),
            pl.BlockSpec((1, H), lambda i, b: (0, 0)),
            pl.BlockSpec((1, H), lambda i, b: (0, 0)),
        ],
        out_specs=pl.BlockSpec((1, _BS, H), lambda i, b: (b, i, 0)),
        out_shape=jax.ShapeDtypeStruct((B, S, H), jnp.float32),
    )(
        inputs_embeds,
        position_table[:S],
        ln_gamma.reshape(1, H),
        ln_beta.reshape(1, H),
    )


# speedup vs baseline: 2.3533x; 1.1833x over previous
import jax
import jax.numpy as jnp
from jax.experimental import pallas as pl

_EPS = 1e-12
_BS = 1024


def _ln_body(in_ref, pos_ref, g_ref, b_ref, out_ref):
    x = in_ref[0] + pos_ref[...]
    mean = jnp.mean(x, axis=-1, keepdims=True)
    xc = x - mean
    var = jnp.mean(xc * xc, axis=-1, keepdims=True)
    normed = xc * jax.lax.rsqrt(var + _EPS)
    out_ref[0] = normed * g_ref[0] + b_ref[0]


def kernel(inputs_embeds, position_table, ln_gamma, ln_beta):
    B, S, H = inputs_embeds.shape
    grid = (S // _BS, B)
    return pl.pallas_call(
        _ln_body,
        grid=grid,
        in_specs=[
            pl.BlockSpec((1, _BS, H), lambda i, b: (b, i, 0)),
            pl.BlockSpec((_BS, H), lambda i, b: (i, 0)),
            pl.BlockSpec((1, H), lambda i, b: (0, 0)),
            pl.BlockSpec((1, H), lambda i, b: (0, 0)),
        ],
        out_specs=pl.BlockSpec((1, _BS, H), lambda i, b: (b, i, 0)),
        out_shape=jax.ShapeDtypeStruct((B, S, H), jnp.float32),
    )(
        inputs_embeds,
        position_table[:S],
        ln_gamma.reshape(1, H),
        ln_beta.reshape(1, H),
    )


# speedup vs baseline: 2.5584x; 1.0872x over previous
import jax
import jax.numpy as jnp
from jax.experimental import pallas as pl

_EPS = 1e-12
_BS = 2048


def _ln_body(in_ref, pos_ref, g_ref, b_ref, out_ref):
    x = in_ref[0] + pos_ref[...]
    mean = jnp.mean(x, axis=-1, keepdims=True)
    xc = x - mean
    var = jnp.mean(xc * xc, axis=-1, keepdims=True)
    normed = xc * jax.lax.rsqrt(var + _EPS)
    out_ref[0] = normed * g_ref[0] + b_ref[0]


def kernel(inputs_embeds, position_table, ln_gamma, ln_beta):
    B, S, H = inputs_embeds.shape
    grid = (S // _BS, B)
    return pl.pallas_call(
        _ln_body,
        grid=grid,
        in_specs=[
            pl.BlockSpec((1, _BS, H), lambda i, b: (b, i, 0)),
            pl.BlockSpec((_BS, H), lambda i, b: (i, 0)),
            pl.BlockSpec((1, H), lambda i, b: (0, 0)),
            pl.BlockSpec((1, H), lambda i, b: (0, 0)),
        ],
        out_specs=pl.BlockSpec((1, _BS, H), lambda i, b: (b, i, 0)),
        out_shape=jax.ShapeDtypeStruct((B, S, H), jnp.float32),
    )(
        inputs_embeds,
        position_table[:S],
        ln_gamma.reshape(1, H),
        ln_beta.reshape(1, H),
    )
